# baseline (speedup 1.0000x reference)
import jax
import jax.numpy as jnp
from jax import lax
from jax.experimental import pallas as pl
from jax.experimental.pallas import tpu as pltpu

BATCH = 128
NCAT = 100000
RB = 8
NRB = BATCH // RB

CW = 1280
NFULL = NCAT // CW
TAIL = NCAT - NFULL * CW

_K2 = 1234
_KS2 = _K2 ^ 0x1BD11BDA
_ROT0 = (13, 15, 26, 6)
_ROT1 = (17, 29, 16, 24)


def _rotl(x, d):
    return (x << jnp.uint32(d)) | (x >> jnp.uint32(32 - d))


def _threefry_bits(x1):
    x0 = x1
    x1 = _rotl(x1, 13) ^ x0
    for r in _ROT0[1:]:
        x0 = x0 + x1
        x1 = _rotl(x1, r) ^ x0
    x0 = x0 + jnp.uint32(_K2)
    x1 = x1 + jnp.uint32(_KS2 + 1)
    for r in _ROT1:
        x0 = x0 + x1
        x1 = _rotl(x1, r) ^ x0
    x0 = x0 + jnp.uint32(_KS2)
    x1 = x1 + jnp.uint32(2)
    for r in _ROT0:
        x0 = x0 + x1
        x1 = _rotl(x1, r) ^ x0
    x1 = x1 + jnp.uint32(_K2 + 3)
    for r in _ROT1:
        x0 = x0 + x1
        x1 = _rotl(x1, r) ^ x0
    x0 = x0 + jnp.uint32(_K2)
    x1 = x1 + jnp.uint32(_KS2 + 4)
    for r in _ROT0:
        x0 = x0 + x1
        x1 = _rotl(x1, r) ^ x0
    x0 = x0 + jnp.uint32(_KS2)
    x1 = x1 + jnp.uint32(5)
    return x0 ^ x1


def _gumbel_from_bits(bits):
    fb = (bits >> jnp.uint32(9)) | jnp.uint32(0x3F800000)
    floats = lax.bitcast_convert_type(fb, jnp.float32) - jnp.float32(1.0)
    u = jnp.maximum(jnp.float32(1.1754943508222875e-38), floats)
    return -jnp.log(-jnp.log(u))


def _body(x_ref, out_ref, idx_scr):
    s = pl.program_id(0)

    @pl.when(s < NRB)
    def _argmax():
        row = s * RB + lax.broadcasted_iota(jnp.int32, (RB, CW), 0)
        basep = row * NCAT + jnp.int32(_K2)
        col0 = lax.broadcasted_iota(jnp.int32, (RB, CW), 1)

        def body(j, carry):
            run_z, run_c = carry
            col = j * CW + col0
            x1 = (basep + col).astype(jnp.uint32)
            x = x_ref[:, pl.ds(j * CW, CW)]
            z = x + _gumbel_from_bits(_threefry_bits(x1))
            better = z > run_z
            return (jnp.where(better, z, run_z), jnp.where(better, col, run_c))

        init = (jnp.full((RB, CW), -jnp.inf, jnp.float32),
                jnp.zeros((RB, CW), jnp.int32))
        run_z, run_c = lax.fori_loop(0, NFULL, body, init)

        colt = NFULL * CW + lax.broadcasted_iota(jnp.int32, (RB, TAIL), 1)
        rowt = s * RB + lax.broadcasted_iota(jnp.int32, (RB, TAIL), 0)
        x1t = (rowt * NCAT + jnp.int32(_K2) + colt).astype(jnp.uint32)
        xt = x_ref[:, pl.ds(NFULL * CW, TAIL)]
        zt = xt + _gumbel_from_bits(_threefry_bits(x1t))

        big = jnp.int32(2**31 - 1)
        rmax = jnp.maximum(jnp.max(run_z, axis=1, keepdims=True),
                           jnp.max(zt, axis=1, keepdims=True))
        cand = jnp.min(jnp.where(run_z == rmax, run_c, big),
                       axis=1, keepdims=True)
        candt = jnp.min(jnp.where(zt == rmax, colt, big),
                        axis=1, keepdims=True)
        idx_scr[pl.ds(s * RB, RB), :] = jnp.minimum(cand, candt)

    @pl.when(s > 0)
    def _onehot():
        idx = idx_scr[pl.ds((s - 1) * RB, RB), :]
        col = lax.broadcasted_iota(jnp.int32, (RB, NCAT), 1)
        out_ref[...] = (col == idx).astype(jnp.float32)


@jax.jit
def kernel(dist_params):
    return pl.pallas_call(
        _body,
        grid=(NRB + 1,),
        in_specs=[pl.BlockSpec((RB, NCAT), lambda s: (jnp.minimum(s, NRB - 1), 0))],
        out_specs=pl.BlockSpec((RB, NCAT), lambda s: (jnp.maximum(s - 1, 0), 0)),
        out_shape=jax.ShapeDtypeStruct((BATCH, NCAT), jnp.float32),
        scratch_shapes=[pltpu.VMEM((BATCH, 1), jnp.int32)],
    )(dist_params)


# speedup vs baseline: 1.0180x; 1.0180x over previous
import jax
import jax.numpy as jnp
from jax import lax
from jax.experimental import pallas as pl
from jax.experimental.pallas import tpu as pltpu

BATCH = 128
NCAT = 100000
RB = 8
NRB = BATCH // RB

CW = 1536
NFULL = NCAT // CW
TAIL = NCAT - NFULL * CW

_K2 = 1234
_KS2 = _K2 ^ 0x1BD11BDA
_ROT0 = (13, 15, 26, 6)
_ROT1 = (17, 29, 16, 24)


def _rotl(x, d):
    return (x << jnp.uint32(d)) | (x >> jnp.uint32(32 - d))


def _threefry_bits(x1):
    x0 = x1
    x1 = _rotl(x1, 13) ^ x0
    for r in _ROT0[1:]:
        x0 = x0 + x1
        x1 = _rotl(x1, r) ^ x0
    x0 = x0 + jnp.uint32(_K2)
    x1 = x1 + jnp.uint32(_KS2 + 1)
    for r in _ROT1:
        x0 = x0 + x1
        x1 = _rotl(x1, r) ^ x0
    x0 = x0 + jnp.uint32(_KS2)
    x1 = x1 + jnp.uint32(2)
    for r in _ROT0:
        x0 = x0 + x1
        x1 = _rotl(x1, r) ^ x0
    x1 = x1 + jnp.uint32(_K2 + 3)
    for r in _ROT1:
        x0 = x0 + x1
        x1 = _rotl(x1, r) ^ x0
    x0 = x0 + jnp.uint32(_K2)
    x1 = x1 + jnp.uint32(_KS2 + 4)
    for r in _ROT0:
        x0 = x0 + x1
        x1 = _rotl(x1, r) ^ x0
    x0 = x0 + jnp.uint32(_KS2)
    x1 = x1 + jnp.uint32(5)
    return x0 ^ x1


def _gumbel_from_bits(bits):
    fb = (bits >> jnp.uint32(9)) | jnp.uint32(0x3F800000)
    floats = lax.bitcast_convert_type(fb, jnp.float32) - jnp.float32(1.0)
    u = jnp.maximum(jnp.float32(1.1754943508222875e-38), floats)
    return -jnp.log(-jnp.log(u))


def _body(x_ref, out_ref, idx_scr):
    s = pl.program_id(0)

    @pl.when(s < NRB)
    def _argmax():
        row = s * RB + lax.broadcasted_iota(jnp.int32, (RB, CW), 0)
        basep = row * NCAT + jnp.int32(_K2)
        col0 = lax.broadcasted_iota(jnp.int32, (RB, CW), 1)

        def body(j, carry):
            run_z, run_c = carry
            col = j * CW + col0
            x1 = (basep + col).astype(jnp.uint32)
            x = x_ref[:, pl.ds(j * CW, CW)]
            z = x + _gumbel_from_bits(_threefry_bits(x1))
            better = z > run_z
            return (jnp.where(better, z, run_z), jnp.where(better, col, run_c))

        init = (jnp.full((RB, CW), -jnp.inf, jnp.float32),
                jnp.zeros((RB, CW), jnp.int32))
        run_z, run_c = lax.fori_loop(0, NFULL, body, init)

        colt = NFULL * CW + lax.broadcasted_iota(jnp.int32, (RB, TAIL), 1)
        rowt = s * RB + lax.broadcasted_iota(jnp.int32, (RB, TAIL), 0)
        x1t = (rowt * NCAT + jnp.int32(_K2) + colt).astype(jnp.uint32)
        xt = x_ref[:, pl.ds(NFULL * CW, TAIL)]
        zt = xt + _gumbel_from_bits(_threefry_bits(x1t))

        big = jnp.int32(2**31 - 1)
        rmax = jnp.maximum(jnp.max(run_z, axis=1, keepdims=True),
                           jnp.max(zt, axis=1, keepdims=True))
        cand = jnp.min(jnp.where(run_z == rmax, run_c, big),
                       axis=1, keepdims=True)
        candt = jnp.min(jnp.where(zt == rmax, colt, big),
                        axis=1, keepdims=True)
        idx_scr[pl.ds(s * RB, RB), :] = jnp.minimum(cand, candt)

    @pl.when(s > 0)
    def _onehot():
        idx = idx_scr[pl.ds((s - 1) * RB, RB), :]
        col = lax.broadcasted_iota(jnp.int32, (RB, NCAT), 1)
        out_ref[...] = (col == idx).astype(jnp.float32)


@jax.jit
def kernel(dist_params):
    return pl.pallas_call(
        _body,
        grid=(NRB + 1,),
        in_specs=[pl.BlockSpec((RB, NCAT), lambda s: (jnp.minimum(s, NRB - 1), 0))],
        out_specs=pl.BlockSpec((RB, NCAT), lambda s: (jnp.maximum(s - 1, 0), 0)),
        out_shape=jax.ShapeDtypeStruct((BATCH, NCAT), jnp.float32),
        scratch_shapes=[pltpu.VMEM((BATCH, 1), jnp.int32)],
    )(dist_params)


# speedup vs baseline: 1.0220x; 1.0039x over previous
import jax
import jax.numpy as jnp
from jax import lax
from jax.experimental import pallas as pl
from jax.experimental.pallas import tpu as pltpu

BATCH = 128
NCAT = 100000
RB = 8
NRB = BATCH // RB

CW = 1664
NFULL = NCAT // CW
TAIL = NCAT - NFULL * CW

_K2 = 1234
_KS2 = _K2 ^ 0x1BD11BDA
_ROT0 = (13, 15, 26, 6)
_ROT1 = (17, 29, 16, 24)


def _rotl(x, d):
    return (x << jnp.uint32(d)) | (x >> jnp.uint32(32 - d))


def _threefry_bits(x1):
    x0 = x1
    x1 = _rotl(x1, 13) ^ x0
    for r in _ROT0[1:]:
        x0 = x0 + x1
        x1 = _rotl(x1, r) ^ x0
    x0 = x0 + jnp.uint32(_K2)
    x1 = x1 + jnp.uint32(_KS2 + 1)
    for r in _ROT1:
        x0 = x0 + x1
        x1 = _rotl(x1, r) ^ x0
    x0 = x0 + jnp.uint32(_KS2)
    x1 = x1 + jnp.uint32(2)
    for r in _ROT0:
        x0 = x0 + x1
        x1 = _rotl(x1, r) ^ x0
    x1 = x1 + jnp.uint32(_K2 + 3)
    for r in _ROT1:
        x0 = x0 + x1
        x1 = _rotl(x1, r) ^ x0
    x0 = x0 + jnp.uint32(_K2)
    x1 = x1 + jnp.uint32(_KS2 + 4)
    for r in _ROT0:
        x0 = x0 + x1
        x1 = _rotl(x1, r) ^ x0
    x0 = x0 + jnp.uint32(_KS2)
    x1 = x1 + jnp.uint32(5)
    return x0 ^ x1


def _gumbel_from_bits(bits):
    fb = (bits >> jnp.uint32(9)) | jnp.uint32(0x3F800000)
    floats = lax.bitcast_convert_type(fb, jnp.float32) - jnp.float32(1.0)
    u = jnp.maximum(jnp.float32(1.1754943508222875e-38), floats)
    return -jnp.log(-jnp.log(u))


def _body(x_ref, out_ref, idx_scr):
    s = pl.program_id(0)

    @pl.when(s < NRB)
    def _argmax():
        row = s * RB + lax.broadcasted_iota(jnp.int32, (RB, CW), 0)
        basep = row * NCAT + jnp.int32(_K2)
        col0 = lax.broadcasted_iota(jnp.int32, (RB, CW), 1)

        def body(j, carry):
            run_z, run_c = carry
            col = j * CW + col0
            x1 = (basep + col).astype(jnp.uint32)
            x = x_ref[:, pl.ds(j * CW, CW)]
            z = x + _gumbel_from_bits(_threefry_bits(x1))
            better = z > run_z
            return (jnp.where(better, z, run_z), jnp.where(better, col, run_c))

        init = (jnp.full((RB, CW), -jnp.inf, jnp.float32),
                jnp.zeros((RB, CW), jnp.int32))
        run_z, run_c = lax.fori_loop(0, NFULL, body, init)

        colt = NFULL * CW + lax.broadcasted_iota(jnp.int32, (RB, TAIL), 1)
        rowt = s * RB + lax.broadcasted_iota(jnp.int32, (RB, TAIL), 0)
        x1t = (rowt * NCAT + jnp.int32(_K2) + colt).astype(jnp.uint32)
        xt = x_ref[:, pl.ds(NFULL * CW, TAIL)]
        zt = xt + _gumbel_from_bits(_threefry_bits(x1t))

        big = jnp.int32(2**31 - 1)
        rmax = jnp.maximum(jnp.max(run_z, axis=1, keepdims=True),
                           jnp.max(zt, axis=1, keepdims=True))
        cand = jnp.min(jnp.where(run_z == rmax, run_c, big),
                       axis=1, keepdims=True)
        candt = jnp.min(jnp.where(zt == rmax, colt, big),
                        axis=1, keepdims=True)
        idx_scr[pl.ds(s * RB, RB), :] = jnp.minimum(cand, candt)

    @pl.when(s > 0)
    def _onehot():
        idx = idx_scr[pl.ds((s - 1) * RB, RB), :]
        col = lax.broadcasted_iota(jnp.int32, (RB, NCAT), 1)
        out_ref[...] = (col == idx).astype(jnp.float32)


@jax.jit
def kernel(dist_params):
    return pl.pallas_call(
        _body,
        grid=(NRB + 1,),
        in_specs=[pl.BlockSpec((RB, NCAT), lambda s: (jnp.minimum(s, NRB - 1), 0))],
        out_specs=pl.BlockSpec((RB, NCAT), lambda s: (jnp.maximum(s - 1, 0), 0)),
        out_shape=jax.ShapeDtypeStruct((BATCH, NCAT), jnp.float32),
        scratch_shapes=[pltpu.VMEM((BATCH, 1), jnp.int32)],
    )(dist_params)


# speedup vs baseline: 1.0315x; 1.0093x over previous
import jax
import jax.numpy as jnp
from jax import lax
from jax.experimental import pallas as pl
from jax.experimental.pallas import tpu as pltpu

BATCH = 128
NCAT = 100000
RB = 8
NRB = BATCH // RB

CW = 1920
NFULL = NCAT // CW
TAIL = NCAT - NFULL * CW

_K2 = 1234
_KS2 = _K2 ^ 0x1BD11BDA
_ROT0 = (13, 15, 26, 6)
_ROT1 = (17, 29, 16, 24)


def _rotl(x, d):
    return (x << jnp.uint32(d)) | (x >> jnp.uint32(32 - d))


def _threefry_bits(x1):
    x0 = x1
    x1 = _rotl(x1, 13) ^ x0
    for r in _ROT0[1:]:
        x0 = x0 + x1
        x1 = _rotl(x1, r) ^ x0
    x0 = x0 + jnp.uint32(_K2)
    x1 = x1 + jnp.uint32(_KS2 + 1)
    for r in _ROT1:
        x0 = x0 + x1
        x1 = _rotl(x1, r) ^ x0
    x0 = x0 + jnp.uint32(_KS2)
    x1 = x1 + jnp.uint32(2)
    for r in _ROT0:
        x0 = x0 + x1
        x1 = _rotl(x1, r) ^ x0
    x1 = x1 + jnp.uint32(_K2 + 3)
    for r in _ROT1:
        x0 = x0 + x1
        x1 = _rotl(x1, r) ^ x0
    x0 = x0 + jnp.uint32(_K2)
    x1 = x1 + jnp.uint32(_KS2 + 4)
    for r in _ROT0:
        x0 = x0 + x1
        x1 = _rotl(x1, r) ^ x0
    x0 = x0 + jnp.uint32(_KS2)
    x1 = x1 + jnp.uint32(5)
    return x0 ^ x1


def _gumbel_from_bits(bits):
    fb = (bits >> jnp.uint32(9)) | jnp.uint32(0x3F800000)
    floats = lax.bitcast_convert_type(fb, jnp.float32) - jnp.float32(1.0)
    u = jnp.maximum(jnp.float32(1.1754943508222875e-38), floats)
    return -jnp.log(-jnp.log(u))


def _body(x_ref, out_ref, idx_scr):
    s = pl.program_id(0)

    @pl.when(s < NRB)
    def _argmax():
        row = s * RB + lax.broadcasted_iota(jnp.int32, (RB, CW), 0)
        basep = row * NCAT + jnp.int32(_K2)
        col0 = lax.broadcasted_iota(jnp.int32, (RB, CW), 1)

        def body(j, carry):
            run_z, run_c = carry
            col = j * CW + col0
            x1 = (basep + col).astype(jnp.uint32)
            x = x_ref[:, pl.ds(j * CW, CW)]
            z = x + _gumbel_from_bits(_threefry_bits(x1))
            better = z > run_z
            return (jnp.where(better, z, run_z), jnp.where(better, col, run_c))

        init = (jnp.full((RB, CW), -jnp.inf, jnp.float32),
                jnp.zeros((RB, CW), jnp.int32))
        run_z, run_c = lax.fori_loop(0, NFULL, body, init)

        colt = NFULL * CW + lax.broadcasted_iota(jnp.int32, (RB, TAIL), 1)
        rowt = s * RB + lax.broadcasted_iota(jnp.int32, (RB, TAIL), 0)
        x1t = (rowt * NCAT + jnp.int32(_K2) + colt).astype(jnp.uint32)
        xt = x_ref[:, pl.ds(NFULL * CW, TAIL)]
        zt = xt + _gumbel_from_bits(_threefry_bits(x1t))

        big = jnp.int32(2**31 - 1)
        rmax = jnp.maximum(jnp.max(run_z, axis=1, keepdims=True),
                           jnp.max(zt, axis=1, keepdims=True))
        cand = jnp.min(jnp.where(run_z == rmax, run_c, big),
                       axis=1, keepdims=True)
        candt = jnp.min(jnp.where(zt == rmax, colt, big),
                        axis=1, keepdims=True)
        idx_scr[pl.ds(s * RB, RB), :] = jnp.minimum(cand, candt)

    @pl.when(s > 0)
    def _onehot():
        idx = idx_scr[pl.ds((s - 1) * RB, RB), :]
        col = lax.broadcasted_iota(jnp.int32, (RB, NCAT), 1)
        out_ref[...] = (col == idx).astype(jnp.float32)


@jax.jit
def kernel(dist_params):
    return pl.pallas_call(
        _body,
        grid=(NRB + 1,),
        in_specs=[pl.BlockSpec((RB, NCAT), lambda s: (jnp.minimum(s, NRB - 1), 0))],
        out_specs=pl.BlockSpec((RB, NCAT), lambda s: (jnp.maximum(s - 1, 0), 0)),
        out_shape=jax.ShapeDtypeStruct((BATCH, NCAT), jnp.float32),
        scratch_shapes=[pltpu.VMEM((BATCH, 1), jnp.int32)],
    )(dist_params)
